# baseline (speedup 1.0000x reference)
import jax
import jax.numpy as jnp
from jax import lax
from jax.experimental import pallas as pl
from jax.experimental.pallas import tpu as pltpu
from jax.experimental.pallas import tpu_sc as plsc

MINUTE_SIZE = 60
D_MODEL = 512

_N = 4096 * 200
_NW = 32
_PER_W = _N // _NW
_CHUNK = 80
_STEPS = _PER_W // _CHUNK
_L = 16
_G = _CHUNK // _L
_DRAIN = 65536
_NDRAIN = _PER_W * D_MODEL // _DRAIN


def _sc_kernel(x_hbm, w_hbm, out_hbm, w_tile, dummy, xbufs, idxs,
               xsems, wsem):
    wid = lax.axis_index("s") * 2 + lax.axis_index("c")
    base0 = wid * _PER_W
    lanes = lax.iota(jnp.int32, _L)

    pltpu.sync_copy(w_hbm, w_tile)

    def stage(g, b):
        base = base0 + g * _CHUNK
        pltpu.async_copy(x_hbm.at[pl.ds(base * 4, _CHUNK * 4)], xbufs[b],
                         xsems[b])

    stage(0, 0)
    stage(1, 1)

    def body(h, carry):
        for b in range(2):
            g = 2 * h + b
            base = base0 + g * _CHUNK
            pltpu.make_async_copy(
                x_hbm.at[pl.ds(0, _CHUNK * 4)], xbufs[b], xsems[b]).wait()
            for j in range(_G):
                flat = lanes * 4 + (j * _L * 4 + 1)
                idxs[b][pl.ds(j * _L, _L)] = (
                    plsc.load_gather(xbufs[b], [flat]) * D_MODEL)

            @pl.when(g + 2 < _STEPS)
            def _():
                stage(g + 2, b)

            @plsc.parallel_loop(0, _G, 1)
            def _(grp):
                ivec = idxs[b][pl.ds(grp * _L, _L)]
                gbase = base + grp * _L
                for l in range(_L):
                    off = pl.multiple_of(ivec[l], D_MODEL)
                    dst = pl.multiple_of((gbase + l) * D_MODEL, D_MODEL)
                    pltpu.async_copy(
                        w_tile.at[pl.ds(off, D_MODEL)],
                        out_hbm.at[pl.ds(dst, D_MODEL)],
                        wsem)

        return carry

    lax.fori_loop(0, _STEPS // 2, body, 0)

    def drain(i, carry):
        pltpu.make_async_copy(
            out_hbm.at[pl.ds(0, _DRAIN)], dummy, wsem).wait()
        return carry

    lax.fori_loop(0, _NDRAIN, drain, 0)


@jax.jit
def kernel(x, W):
    x2 = x.reshape(_N * 4).astype(jnp.int32)
    w2 = W.reshape(MINUTE_SIZE * D_MODEL)
    mesh = plsc.VectorSubcoreMesh(core_axis_name="c", subcore_axis_name="s")

    def body(x_hbm, w_hbm, out_hbm, w_tile, dummy, xb0, xb1, id0, id1,
             xs0, xs1, ws):
        _sc_kernel(x_hbm, w_hbm, out_hbm, w_tile, dummy,
                   (xb0, xb1), (id0, id1), (xs0, xs1), ws)

    out = pl.kernel(
        body,
        mesh=mesh,
        compiler_params=pltpu.CompilerParams(needs_layout_passes=False),
        out_type=jax.ShapeDtypeStruct((_N * D_MODEL,), jnp.float32),
        scratch_types=[
            pltpu.VMEM((MINUTE_SIZE * D_MODEL,), jnp.float32),
            pltpu.VMEM((_DRAIN,), jnp.float32),
            pltpu.VMEM((_CHUNK * 4,), jnp.int32),
            pltpu.VMEM((_CHUNK * 4,), jnp.int32),
            pltpu.VMEM((_CHUNK + _L,), jnp.int32),
            pltpu.VMEM((_CHUNK + _L,), jnp.int32),
            pltpu.SemaphoreType.DMA,
            pltpu.SemaphoreType.DMA,
            pltpu.SemaphoreType.DMA,
        ],
    )(x2, w2)
    return out.reshape(4096, 200, D_MODEL)
